# Initial kernel scaffold; baseline (speedup 1.0000x reference)
import functools

import jax
import jax.numpy as jnp
from jax import lax
from jax.experimental import pallas as pl
from jax.experimental.pallas import tpu as pltpu
from jax.experimental.pallas import tpu_sc as plsc

F32 = jnp.float32
I32 = jnp.int32
G = 128
INNER = 4


def _sc_geom():
    try:
        info = plsc.get_sparse_core_info()
        return int(info.num_cores), int(info.num_subcores)
    except Exception:
        return 2, 16


def _npad(n, ns):
    return ns * (-(-(n + 1) // ns))


def _pad_edges(src, dst, n, nw):
    e = src.shape[0]
    quantum = nw * G * INNER
    epad = quantum * (-(-e // quantum))
    pad = epad - e
    src_p = jnp.concatenate([src, jnp.zeros((pad,), I32)])
    dst_p = jnp.concatenate([dst, jnp.full((pad,), n, I32)])
    return src_p.reshape(-1, G), dst_p.reshape(-1, G)


def _segsum(u, src3, dst3, npad):
    n, c = u.shape
    nc, ns = _sc_geom()
    nw = nc * ns
    ng = src3.shape[0] // nw
    outer = ng // INNER
    rpt = npad // ns
    zeros = jnp.zeros((npad, c), F32)
    mesh = plsc.VectorSubcoreMesh(
        core_axis_name="c", subcore_axis_name="s",
        num_cores=nc, num_subcores=ns)

    @functools.partial(
        pl.kernel,
        out_type=jax.ShapeDtypeStruct((nc, npad, c), F32),
        mesh=mesh,
        scratch_types=[
            pltpu.VMEM((ng, G), I32),
            pltpu.VMEM((ng, G), I32),
            pltpu.VMEM((INNER, G, c), F32),
        ] + [pltpu.SemaphoreType.DMA] * INNER + [
            pltpu.VMEM_SHARED((npad, c), F32),
        ],
    )
    def k(u_hbm, src_hbm, dst_hbm, z_hbm, out_hbm,
          src_v, dst_v, rows_v, s0, s1, s2, s3, acc):
        sems = [s0, s1, s2, s3]
        ci = lax.axis_index("c")
        si = lax.axis_index("s")
        tid = ci * ns + si
        pltpu.sync_copy(z_hbm.at[pl.ds(si * rpt, rpt)],
                        acc.at[pl.ds(si * rpt, rpt)])
        plsc.subcore_barrier()
        g0 = tid * ng
        pltpu.sync_copy(src_hbm.at[pl.ds(g0, ng)], src_v)
        pltpu.sync_copy(dst_hbm.at[pl.ds(g0, ng)], dst_v)
        for b in range(INNER):
            pltpu.async_copy(u_hbm.at[src_v.at[b]], rows_v.at[b], sems[b])

        def obody(o, carry):
            for b in range(INNER):
                g = o * INNER + b
                pltpu.make_async_copy(
                    u_hbm.at[src_v.at[g]], rows_v.at[b], sems[b]).wait()
                pltpu.sync_copy(rows_v.at[b], acc.at[dst_v.at[g]], add=True)

                @pl.when(o < outer - 1)
                def _fire():
                    pltpu.async_copy(
                        u_hbm.at[src_v.at[g + INNER]], rows_v.at[b], sems[b])
            return carry

        lax.fori_loop(0, outer, obody, 0)
        plsc.subcore_barrier()
        pltpu.sync_copy(acc.at[pl.ds(si * rpt, rpt)],
                        out_hbm.at[ci, pl.ds(si * rpt, rpt)])

    return k(u, src3, dst3, zeros)


def _deg(dst3, npad):
    nc, ns = _sc_geom()
    nw = nc * ns
    ng = dst3.shape[0] // nw
    rpt = npad // ns
    ones = jnp.ones((G, 1), F32)
    zeros = jnp.zeros((npad, 1), F32)
    mesh = plsc.VectorSubcoreMesh(
        core_axis_name="c", subcore_axis_name="s",
        num_cores=nc, num_subcores=ns)

    @functools.partial(
        pl.kernel,
        out_type=jax.ShapeDtypeStruct((nc, npad, 1), F32),
        mesh=mesh,
        scratch_types=[
            pltpu.VMEM((ng, G), I32),
            pltpu.VMEM((G, 1), F32),
            pltpu.VMEM_SHARED((npad, 1), F32),
        ],
    )
    def k(dst_hbm, ones_hbm, z_hbm, out_hbm, dst_v, ones_v, acc):
        ci = lax.axis_index("c")
        si = lax.axis_index("s")
        tid = ci * ns + si
        pltpu.sync_copy(z_hbm.at[pl.ds(si * rpt, rpt)],
                        acc.at[pl.ds(si * rpt, rpt)])
        plsc.subcore_barrier()
        pltpu.sync_copy(dst_hbm.at[pl.ds(tid * ng, ng)], dst_v)
        pltpu.sync_copy(ones_hbm, ones_v)

        def obody(g, carry):
            pltpu.sync_copy(ones_v, acc.at[dst_v.at[g]], add=True)
            return carry

        lax.fori_loop(0, ng, obody, 0)
        plsc.subcore_barrier()
        pltpu.sync_copy(acc.at[pl.ds(si * rpt, rpt)],
                        out_hbm.at[ci, pl.ds(si * rpt, rpt)])

    return k(dst3, ones, zeros)


_BLK = 2048


def _prep(p0, p1, a0, a1, scale):
    n, c = a0.shape
    nb = -(-n // _BLK)

    def body(p0_r, p1_r, a0_r, a1_r, dis_r, t0_r, u0_r):
        deg = p0_r[...] + p1_r[...]
        dis = jnp.where(deg > 0, lax.rsqrt(deg), 0.0)
        t0 = scale * (a0_r[...] + a1_r[...])
        dis_r[...] = dis
        t0_r[...] = t0
        u0_r[...] = dis * t0

    return pl.pallas_call(
        body,
        grid=(nb,),
        in_specs=[
            pl.BlockSpec((_BLK, 1), lambda i: (i, 0)),
            pl.BlockSpec((_BLK, 1), lambda i: (i, 0)),
            pl.BlockSpec((_BLK, c), lambda i: (i, 0)),
            pl.BlockSpec((_BLK, c), lambda i: (i, 0)),
        ],
        out_specs=[
            pl.BlockSpec((_BLK, 1), lambda i: (i, 0)),
            pl.BlockSpec((_BLK, c), lambda i: (i, 0)),
            pl.BlockSpec((_BLK, c), lambda i: (i, 0)),
        ],
        out_shape=[
            jax.ShapeDtypeStruct((n, 1), F32),
            jax.ShapeDtypeStruct((n, c), F32),
            jax.ShapeDtypeStruct((n, c), F32),
        ],
    )(p0, p1, a0, a1)


def _combine(s0, s1, dis, tprev, alpha, beta):
    n, c = s0.shape
    nb = -(-n // _BLK)

    def body(s0_r, s1_r, dis_r, tp_r, t_r, u_r):
        t = alpha * (dis_r[...] * (s0_r[...] + s1_r[...])) + beta * tp_r[...]
        t_r[...] = t
        u_r[...] = dis_r[...] * t

    return pl.pallas_call(
        body,
        grid=(nb,),
        in_specs=[
            pl.BlockSpec((_BLK, c), lambda i: (i, 0)),
            pl.BlockSpec((_BLK, c), lambda i: (i, 0)),
            pl.BlockSpec((_BLK, 1), lambda i: (i, 0)),
            pl.BlockSpec((_BLK, c), lambda i: (i, 0)),
        ],
        out_specs=[
            pl.BlockSpec((_BLK, c), lambda i: (i, 0)),
            pl.BlockSpec((_BLK, c), lambda i: (i, 0)),
        ],
        out_shape=[
            jax.ShapeDtypeStruct((n, c), F32),
            jax.ShapeDtypeStruct((n, c), F32),
        ],
    )(s0, s1, dis, tprev)


def _cheb_matmul(ts, w, b, relu):
    n, c = ts[0].shape
    kk, _, cout = w.shape
    nb = -(-n // _BLK)

    def body(*refs):
        t_refs = refs[:kk]
        w_r, b_r, h_r = refs[kk], refs[kk + 1], refs[kk + 2]
        acc = b_r[...].astype(F32) + jnp.zeros((_BLK, cout), F32)
        for k in range(kk):
            acc = acc + jnp.dot(t_refs[k][...], w_r[k],
                                preferred_element_type=F32)
        if relu:
            acc = jnp.maximum(acc, 0.0)
        h_r[...] = acc

    return pl.pallas_call(
        body,
        grid=(nb,),
        in_specs=[pl.BlockSpec((_BLK, c), lambda i: (i, 0))
                  for _ in range(kk)] + [
            pl.BlockSpec((kk, c, cout), lambda i: (0, 0, 0)),
            pl.BlockSpec((1, cout), lambda i: (0, 0)),
        ],
        out_specs=pl.BlockSpec((_BLK, cout), lambda i: (i, 0)),
        out_shape=jax.ShapeDtypeStruct((n, cout), F32),
    )(*ts, w, b.reshape(1, cout))


def _matvec(wlin, hflat, blin):
    ncls, kdim = wlin.shape
    kb = 12500
    nj = kdim // kb

    def body(h_r, w_r, b_r, z_r):
        j = pl.program_id(0)

        @pl.when(j == 0)
        def _init():
            z_r[...] = b_r[...]

        z_r[...] += lax.dot_general(
            h_r[...], w_r[...], (((1,), (1,)), ((), ())),
            preferred_element_type=F32)

    out = pl.pallas_call(
        body,
        grid=(nj,),
        in_specs=[
            pl.BlockSpec((1, kb), lambda j: (0, j)),
            pl.BlockSpec((ncls, kb), lambda j: (0, j)),
            pl.BlockSpec((1, ncls), lambda j: (0, 0)),
        ],
        out_specs=pl.BlockSpec((1, ncls), lambda j: (0, 0)),
        out_shape=jax.ShapeDtypeStruct((1, ncls), F32),
    )(hflat.reshape(1, kdim), wlin, blin.reshape(1, ncls))
    return out.reshape(ncls)


def _cheb_layer(a0, a1, scale, ei, w, b, n, relu):
    nc, ns = _sc_geom()
    nw = nc * ns
    src3, dst3 = _pad_edges(ei[0], ei[1], n, nw)
    npad = _npad(n, ns)
    degp = _deg(dst3, npad)
    dis, t0, u = _prep(degp[0, :n], degp[1, :n], a0, a1, scale)
    ts = [t0]
    kk = w.shape[0]
    for k in range(1, kk):
        sp = _segsum(u, src3, dst3, npad)
        alpha, beta = (-1.0, 0.0) if k == 1 else (-2.0, -1.0)
        tprev = ts[k - 2] if k >= 2 else t0
        t, u = _combine(sp[0, :n], sp[1, :n], dis, tprev, alpha, beta)
        ts.append(t)
    return _cheb_matmul(ts, w, b, relu)


def _pool(h, rows, cols, n_out):
    nc, ns = _sc_geom()
    src3, dst3 = _pad_edges(cols, rows, n_out, nc * ns)
    npad = _npad(n_out, ns)
    sp = _segsum(h, src3, dst3, npad)
    return sp[0, :n_out], sp[1, :n_out]


def kernel(x, ei0, ei1, ei2, d0_rows, d0_cols, d0_vals,
           d1_rows, d1_cols, d1_vals,
           W0, b0, W1, b1, W2, b2, Wlin, blin):
    n0, n1, n2 = 50000, 12500, 3125
    h0 = _cheb_layer(x, x, 0.5, ei0, W0, b0, n0, relu=True)
    a0, a1 = _pool(h0, d0_rows, d0_cols, n1)
    h1 = _cheb_layer(a0, a1, 0.25, ei1, W1, b1, n1, relu=True)
    a0, a1 = _pool(h1, d1_rows, d1_cols, n2)
    h2 = _cheb_layer(a0, a1, 0.25, ei2, W2, b2, n2, relu=False)
    return _matvec(Wlin, h2.reshape(-1), blin)


# speedup vs baseline: 13.0769x; 13.0769x over previous
import functools

import jax
import jax.numpy as jnp
from jax import lax
from jax.experimental import pallas as pl
from jax.experimental.pallas import tpu as pltpu
from jax.experimental.pallas import tpu_sc as plsc

F32 = jnp.float32
I32 = jnp.int32
G = 128
INNER = 4


def _sc_geom():
    try:
        info = plsc.get_sparse_core_info()
        return int(info.num_cores), int(info.num_subcores)
    except Exception:
        return 2, 16


def _npad(n, ns):
    q = 8 * ns
    return q * (-(-(n + 1) // q))


def _pad_edges(src, dst, n, nw):
    e = src.shape[0]
    quantum = nw * G * INNER
    epad = quantum * (-(-e // quantum))
    pad = epad - e
    src_p = jnp.concatenate([src, jnp.zeros((pad,), I32)])
    dst_p = jnp.concatenate([dst, jnp.full((pad,), n, I32)])
    return src_p.reshape(-1, G), dst_p.reshape(-1, G)


def _segsum(u, src3, dst3, npad):
    n, c = u.shape
    nc, ns = _sc_geom()
    nw = nc * ns
    ng = src3.shape[0] // nw
    outer = ng // INNER
    rpt = npad // ns
    zeros = jnp.zeros((npad, c), F32)
    mesh = plsc.VectorSubcoreMesh(
        core_axis_name="c", subcore_axis_name="s",
        num_cores=nc, num_subcores=ns)

    @functools.partial(
        pl.kernel,
        out_type=jax.ShapeDtypeStruct((nc, npad, c), F32),
        mesh=mesh,
        scratch_types=[
            pltpu.VMEM((ng, G), I32),
            pltpu.VMEM((ng, G), I32),
            pltpu.VMEM((INNER, G, c), F32),
        ] + [pltpu.SemaphoreType.DMA] * INNER + [
            pltpu.VMEM_SHARED((npad, c), F32),
        ],
        compiler_params=pltpu.CompilerParams(use_tc_tiling_on_sc=False),
    )
    def k(u_hbm, src_hbm, dst_hbm, z_hbm, out_hbm,
          src_v, dst_v, rows_v, s0, s1, s2, s3, acc):
        sems = [s0, s1, s2, s3]
        ci = lax.axis_index("c")
        si = lax.axis_index("s")
        tid = ci * ns + si
        pltpu.sync_copy(z_hbm.at[pl.ds(si * rpt, rpt)],
                        acc.at[pl.ds(si * rpt, rpt)])
        plsc.subcore_barrier()
        g0 = tid * ng
        pltpu.sync_copy(src_hbm.at[pl.ds(g0, ng)], src_v)
        pltpu.sync_copy(dst_hbm.at[pl.ds(g0, ng)], dst_v)
        for b in range(INNER):
            pltpu.async_copy(u_hbm.at[src_v.at[b]], rows_v.at[b], sems[b])

        def obody(o, carry):
            for b in range(INNER):
                g = o * INNER + b
                pltpu.make_async_copy(
                    u_hbm.at[src_v.at[g]], rows_v.at[b], sems[b]).wait()
                pltpu.sync_copy(rows_v.at[b], acc.at[dst_v.at[g]], add=True)

                @pl.when(o < outer - 1)
                def _fire():
                    pltpu.async_copy(
                        u_hbm.at[src_v.at[g + INNER]], rows_v.at[b], sems[b])
            return carry

        lax.fori_loop(0, outer, obody, 0)
        plsc.subcore_barrier()
        pltpu.sync_copy(acc.at[pl.ds(si * rpt, rpt)],
                        out_hbm.at[ci, pl.ds(si * rpt, rpt)])

    return k(u, src3, dst3, zeros)


_DEGW = 8


def _deg(dst3, npad):
    nc, ns = _sc_geom()
    nw = nc * ns
    ng = dst3.shape[0] // nw
    rpt = npad // ns
    ones = jnp.ones((G, _DEGW), F32)
    zeros = jnp.zeros((npad, _DEGW), F32)
    mesh = plsc.VectorSubcoreMesh(
        core_axis_name="c", subcore_axis_name="s",
        num_cores=nc, num_subcores=ns)

    @functools.partial(
        pl.kernel,
        out_type=jax.ShapeDtypeStruct((nc, npad, _DEGW), F32),
        mesh=mesh,
        scratch_types=[
            pltpu.VMEM((ng, G), I32),
            pltpu.VMEM((G, _DEGW), F32),
            pltpu.VMEM_SHARED((npad, _DEGW), F32),
        ],
        compiler_params=pltpu.CompilerParams(use_tc_tiling_on_sc=False),
    )
    def k(dst_hbm, ones_hbm, z_hbm, out_hbm, dst_v, ones_v, acc):
        ci = lax.axis_index("c")
        si = lax.axis_index("s")
        tid = ci * ns + si
        pltpu.sync_copy(z_hbm.at[pl.ds(si * rpt, rpt)],
                        acc.at[pl.ds(si * rpt, rpt)])
        plsc.subcore_barrier()
        pltpu.sync_copy(dst_hbm.at[pl.ds(tid * ng, ng)], dst_v)
        pltpu.sync_copy(ones_hbm.at[pl.ds(0, G)], ones_v.at[pl.ds(0, G)])

        def obody(g, carry):
            pltpu.sync_copy(ones_v, acc.at[dst_v.at[g]], add=True)
            return carry

        lax.fori_loop(0, ng, obody, 0)
        plsc.subcore_barrier()
        pltpu.sync_copy(acc.at[pl.ds(si * rpt, rpt)],
                        out_hbm.at[ci, pl.ds(si * rpt, rpt)])

    return k(dst3, ones, zeros)


_BLK = 2048


def _prep(p0, p1, a0, a1, scale):
    n, c = a0.shape
    nb = -(-n // _BLK)

    def body(p0_r, p1_r, a0_r, a1_r, dis_r, t0_r, u0_r):
        deg = p0_r[...] + p1_r[...]
        dis = jnp.where(deg > 0, lax.rsqrt(deg), 0.0)
        t0 = scale * (a0_r[...] + a1_r[...])
        dis_r[...] = dis
        t0_r[...] = t0
        u0_r[...] = dis * t0

    return pl.pallas_call(
        body,
        grid=(nb,),
        in_specs=[
            pl.BlockSpec((_BLK, 1), lambda i: (i, 0)),
            pl.BlockSpec((_BLK, 1), lambda i: (i, 0)),
            pl.BlockSpec((_BLK, c), lambda i: (i, 0)),
            pl.BlockSpec((_BLK, c), lambda i: (i, 0)),
        ],
        out_specs=[
            pl.BlockSpec((_BLK, 1), lambda i: (i, 0)),
            pl.BlockSpec((_BLK, c), lambda i: (i, 0)),
            pl.BlockSpec((_BLK, c), lambda i: (i, 0)),
        ],
        out_shape=[
            jax.ShapeDtypeStruct((n, 1), F32),
            jax.ShapeDtypeStruct((n, c), F32),
            jax.ShapeDtypeStruct((n, c), F32),
        ],
    )(p0, p1, a0, a1)


def _combine(s0, s1, dis, tprev, alpha, beta):
    n, c = s0.shape
    nb = -(-n // _BLK)

    def body(s0_r, s1_r, dis_r, tp_r, t_r, u_r):
        t = alpha * (dis_r[...] * (s0_r[...] + s1_r[...])) + beta * tp_r[...]
        t_r[...] = t
        u_r[...] = dis_r[...] * t

    return pl.pallas_call(
        body,
        grid=(nb,),
        in_specs=[
            pl.BlockSpec((_BLK, c), lambda i: (i, 0)),
            pl.BlockSpec((_BLK, c), lambda i: (i, 0)),
            pl.BlockSpec((_BLK, 1), lambda i: (i, 0)),
            pl.BlockSpec((_BLK, c), lambda i: (i, 0)),
        ],
        out_specs=[
            pl.BlockSpec((_BLK, c), lambda i: (i, 0)),
            pl.BlockSpec((_BLK, c), lambda i: (i, 0)),
        ],
        out_shape=[
            jax.ShapeDtypeStruct((n, c), F32),
            jax.ShapeDtypeStruct((n, c), F32),
        ],
    )(s0, s1, dis, tprev)


def _cheb_matmul(ts, w, b, relu):
    n, c = ts[0].shape
    kk, _, cout = w.shape
    nb = -(-n // _BLK)

    def body(*refs):
        t_refs = refs[:kk]
        w_r, b_r, h_r = refs[kk], refs[kk + 1], refs[kk + 2]
        acc = b_r[...].astype(F32) + jnp.zeros((_BLK, cout), F32)
        for k in range(kk):
            acc = acc + jnp.dot(t_refs[k][...], w_r[k],
                                preferred_element_type=F32)
        if relu:
            acc = jnp.maximum(acc, 0.0)
        h_r[...] = acc

    return pl.pallas_call(
        body,
        grid=(nb,),
        in_specs=[pl.BlockSpec((_BLK, c), lambda i: (i, 0))
                  for _ in range(kk)] + [
            pl.BlockSpec((kk, c, cout), lambda i: (0, 0, 0)),
            pl.BlockSpec((1, cout), lambda i: (0, 0)),
        ],
        out_specs=pl.BlockSpec((_BLK, cout), lambda i: (i, 0)),
        out_shape=jax.ShapeDtypeStruct((n, cout), F32),
    )(*ts, w, b.reshape(1, cout))


def _matvec(wlin, hflat, blin):
    ncls, kdim = wlin.shape
    cb = 8
    ni = ncls // cb

    def body(h_r, w_r, b_r, z_r):
        z_r[...] = b_r[...] + lax.dot_general(
            w_r[...], h_r[...], (((1,), (1,)), ((), ())),
            preferred_element_type=F32)

    out = pl.pallas_call(
        body,
        grid=(ni,),
        in_specs=[
            pl.BlockSpec((1, kdim), lambda i: (0, 0)),
            pl.BlockSpec((cb, kdim), lambda i: (i, 0)),
            pl.BlockSpec((cb, 1), lambda i: (i, 0)),
        ],
        out_specs=pl.BlockSpec((cb, 1), lambda i: (i, 0)),
        out_shape=jax.ShapeDtypeStruct((ncls, 1), F32),
    )(hflat.reshape(1, kdim), wlin, blin.reshape(ncls, 1))
    return out.reshape(ncls)


def _cheb_layer(a0, a1, scale, ei, w, b, n, relu):
    nc, ns = _sc_geom()
    nw = nc * ns
    src3, dst3 = _pad_edges(ei[0], ei[1], n, nw)
    npad = _npad(n, ns)
    degp = _deg(dst3, npad)
    dis, t0, u = _prep(degp[0, :n, :1], degp[1, :n, :1], a0, a1, scale)
    ts = [t0]
    kk = w.shape[0]
    for k in range(1, kk):
        sp = _segsum(u, src3, dst3, npad)
        alpha, beta = (-1.0, 0.0) if k == 1 else (-2.0, -1.0)
        tprev = ts[k - 2] if k >= 2 else t0
        t, u = _combine(sp[0, :n], sp[1, :n], dis, tprev, alpha, beta)
        ts.append(t)
    return _cheb_matmul(ts, w, b, relu)


def _pool(h, rows, cols, n_out):
    nc, ns = _sc_geom()
    src3, dst3 = _pad_edges(cols, rows, n_out, nc * ns)
    npad = _npad(n_out, ns)
    sp = _segsum(h, src3, dst3, npad)
    return sp[0, :n_out], sp[1, :n_out]


def kernel(x, ei0, ei1, ei2, d0_rows, d0_cols, d0_vals,
           d1_rows, d1_cols, d1_vals,
           W0, b0, W1, b1, W2, b2, Wlin, blin):
    n0, n1, n2 = 50000, 12500, 3125
    x8 = jnp.pad(x, ((0, 0), (0, 5)))
    w0p = jnp.pad(W0, ((0, 0), (0, 5), (0, 0)))
    h0 = _cheb_layer(x8, x8, 0.5, ei0, w0p, b0, n0, relu=True)
    a0, a1 = _pool(h0, d0_rows, d0_cols, n1)
    h1 = _cheb_layer(a0, a1, 0.25, ei1, W1, b1, n1, relu=True)
    a0, a1 = _pool(h1, d1_rows, d1_cols, n2)
    h2 = _cheb_layer(a0, a1, 0.25, ei2, W2, b2, n2, relu=False)
    return _matvec(Wlin, h2.reshape(-1), blin)
